# baseline (speedup 1.0000x reference)
import functools

import jax
import jax.numpy as jnp
from jax import lax
from jax.experimental import pallas as pl
from jax.experimental.pallas import tpu as pltpu
from jax.experimental.pallas import tpu_sc as plsc

N_CLS = 1000
CTX_LEN = 77
VOCAB = 49408
D = 768
N_CTX = 8
LANES = 128
DB = D // LANES
CPB = 40
KC = N_CLS // CPB
UNIT = CPB * DB
HALF = UNIT // 2
UNITS = CTX_LEN * KC
NW = 32
UPW = (UNITS + NW - 1) // NW
UNITS_PAD = NW * UPW
CBLK = 8 * DB
NBUF = 3

_mesh = plsc.VectorSubcoreMesh(core_axis_name="c", subcore_axis_name="s")


@functools.partial(
    pl.kernel,
    out_type=jax.ShapeDtypeStruct((CTX_LEN * KC * UNIT, LANES), jnp.float32),
    mesh=_mesh,
    scratch_types=[
        pltpu.VMEM((UPW, 2, HALF), jnp.int32),
        pltpu.VMEM((NBUF, UNIT, LANES), jnp.float32),
        pltpu.SemaphoreType.DMA,
        pltpu.SemaphoreType.DMA,
    ],
    compiler_params=pltpu.CompilerParams(use_tc_tiling_on_sc=False),
)
def _prompt_gather(table_c, ctx_c, gidx, out, gidx_v, bufs, gsem, wsem):
    w = lax.axis_index("s") * 2 + lax.axis_index("c")
    n_u = UNITS // NW + jnp.where(w < UNITS % NW, 1, 0)
    pltpu.sync_copy(gidx.at[pl.ds(w * UPW, UPW)], gidx_v)

    def unit_u(j):
        return j * NW + w

    def unit_t(j):
        return unit_u(j) // KC

    def is_ctx(t):
        return (t >= 1) & (t <= 1 + N_CTX - 1)

    def start_gathers(j):
        b = j % NBUF
        t = unit_t(j)

        @pl.when(is_ctx(t))
        def _():
            pltpu.async_copy(
                ctx_c.at[gidx_v.at[j, 0, pl.ds(0, CBLK)]],
                bufs.at[b, pl.ds(0, CBLK)], gsem)

        @pl.when(~is_ctx(t))
        def _():
            pltpu.async_copy(
                table_c.at[gidx_v.at[j, 0]], bufs.at[b, pl.ds(0, HALF)], gsem)
            pltpu.async_copy(
                table_c.at[gidx_v.at[j, 1]], bufs.at[b, pl.ds(HALF, HALF)], gsem)

    def wait_gathers(j):
        b = j % NBUF
        t = unit_t(j)

        @pl.when(is_ctx(t))
        def _():
            pltpu.make_async_copy(
                ctx_c.at[gidx_v.at[j, 0, pl.ds(0, CBLK)]],
                bufs.at[b, pl.ds(0, CBLK)], gsem
            ).wait()

        @pl.when(~is_ctx(t))
        def _():
            pltpu.make_async_copy(
                table_c.at[gidx_v.at[j, 0]], bufs.at[b, pl.ds(0, HALF)], gsem
            ).wait()
            pltpu.make_async_copy(
                table_c.at[gidx_v.at[j, 1]], bufs.at[b, pl.ds(HALF, HALF)], gsem
            ).wait()

    def wait_write(j):
        pltpu.make_async_copy(
            bufs.at[j % NBUF], out.at[pl.ds(unit_u(j) * UNIT, UNIT)], wsem
        ).wait()

    def body(j, carry):
        @pl.when(j < n_u)
        def _():
            @pl.when(j == 0)
            def _():
                start_gathers(0)

                @pl.when(n_u > 1)
                def _():
                    start_gathers(1)

            wait_gathers(j)

            @pl.when(j + 2 < n_u)
            def _():
                @pl.when(j + 2 >= NBUF)
                def _():
                    wait_write(j + 2 - NBUF)

                start_gathers(j + 2)

            @pl.when(is_ctx(unit_t(j)))
            def _():
                for m in range(UNIT // CBLK):
                    pltpu.async_copy(
                        bufs.at[j % NBUF, pl.ds(0, CBLK)],
                        out.at[pl.ds(unit_u(j) * UNIT + m * CBLK, CBLK)],
                        wsem,
                    )

            @pl.when(~is_ctx(unit_t(j)))
            def _():
                pltpu.async_copy(
                    bufs.at[j % NBUF],
                    out.at[pl.ds(unit_u(j) * UNIT, UNIT)],
                    wsem,
                )

        return carry

    lax.fori_loop(0, UPW, body, 0)

    def drain(m, carry):
        j = n_u - NBUF + m

        @pl.when(j >= 0)
        def _():
            wait_write(j)

        return carry

    lax.fori_loop(0, NBUF, drain, 0)


def kernel(token_embedding_weight, ctx, tokenized_prompts):
    table_c = (
        token_embedding_weight.reshape(VOCAB // 8, 8, DB, LANES)
        .transpose(0, 2, 1, 3)
        .reshape(VOCAB * DB, LANES)
    )
    ctx_c = (
        ctx.reshape(1, N_CTX, DB, LANES)
        .transpose(0, 2, 1, 3)
        .reshape(N_CTX * DB, LANES)
    )

    rt = tokenized_prompts.T.reshape(CTX_LEN, KC, CPB // 8, 8)
    base = (rt // 8) * (8 * DB) + (rt % 8)
    dmul = (jnp.arange(DB, dtype=jnp.int32) * 8)[None, None, None, :, None]
    gidx = (base[:, :, :, None, :] + dmul).reshape(CTX_LEN, KC, UNIT)
    cpat = (
        jnp.arange(N_CTX, dtype=jnp.int32)[:, None, None, None]
        + (jnp.arange(DB, dtype=jnp.int32) * 8)[None, None, :, None]
        + jnp.zeros((N_CTX, CPB // 8, DB, 8), jnp.int32)
    ).reshape(N_CTX, UNIT)
    gidx = gidx.at[1:1 + N_CTX].set(cpat[:, None, :])
    gidx = gidx.reshape(UNITS, 2, HALF)
    gidx = jnp.pad(gidx, ((0, UNITS_PAD - UNITS), (0, 0), (0, 0)))
    gidx = gidx.reshape(UPW, NW, 2, HALF).transpose(1, 0, 2, 3)
    gidx = gidx.reshape(UNITS_PAD, 2, HALF)

    out = _prompt_gather(table_c, ctx_c, gidx)
    return (
        out.reshape(CTX_LEN, N_CLS // 8, DB, 8, LANES)
        .transpose(1, 3, 0, 2, 4)
        .reshape(N_CLS, CTX_LEN, D)
    )
